# baseline (speedup 1.0000x reference)
import functools

import jax
import jax.numpy as jnp
from jax import lax
from jax.experimental import pallas as pl
from jax.experimental.pallas import tpu as pltpu
from jax.experimental.pallas import tpu_sc as plsc

NC, NS = 2, 16
NW = NC * NS
K = 80
NP = 10240
INV_BN = 1.0 / (1.0 + 1e-5) ** 0.5


def _gather_pair(A, B, dstr, srcr):
    N, h = A.shape
    dt = A.dtype
    _, CH, _ = dstr.shape
    E = NW * CH * K
    EW = CH * K

    NB = 5
    FULL = CH // NB
    TAIL = range(FULL * NB, CH)

    @functools.partial(
        pl.kernel,
        out_type=[jax.ShapeDtypeStruct((E, h), dt),
                  jax.ShapeDtypeStruct((E, h), dt)],
        mesh=plsc.VectorSubcoreMesh(core_axis_name="c", subcore_axis_name="s"),
        scratch_types=[pltpu.VMEM((CH, K), jnp.int32),
                       pltpu.VMEM((CH, K), jnp.int32),
                       pltpu.VMEM((NB, K, h), dt),
                       pltpu.VMEM((NB, K, h), dt)]
                      + [pltpu.SemaphoreType.DMA] * (4 * NB),
    )
    def gk(a_hbm, b_hbm, d_hbm, s_hbm, ad_hbm, bs_hbm, idxd, idxs, bufa, bufb,
           *sems):
        sga, sgb, sta, stb = (sems[:NB], sems[NB:2 * NB], sems[2 * NB:3 * NB],
                              sems[3 * NB:])
        w = lax.axis_index("s") * NC + lax.axis_index("c")
        pltpu.sync_copy(d_hbm.at[w], idxd)
        pltpu.sync_copy(s_hbm.at[w], idxs)
        for b in range(NB):
            pltpu.async_copy(a_hbm.at[idxd.at[b]], bufa.at[b], sga[b])
            pltpu.async_copy(b_hbm.at[idxs.at[b]], bufb.at[b], sgb[b])

        def step(c, b, base):
            pltpu.make_async_copy(a_hbm.at[idxd.at[c]], bufa.at[b],
                                  sga[b]).wait()
            pltpu.make_async_copy(b_hbm.at[idxs.at[c]], bufb.at[b],
                                  sgb[b]).wait()
            pltpu.async_copy(bufa.at[b], ad_hbm.at[pl.ds(base, K)], sta[b])
            pltpu.async_copy(bufb.at[b], bs_hbm.at[pl.ds(base, K)], stb[b])

            @pl.when(c + NB < CH)
            def _():
                pltpu.make_async_copy(bufa.at[b], ad_hbm.at[pl.ds(base, K)],
                                      sta[b]).wait()
                pltpu.make_async_copy(bufb.at[b], bs_hbm.at[pl.ds(base, K)],
                                      stb[b]).wait()
                pltpu.async_copy(a_hbm.at[idxd.at[c + NB]], bufa.at[b], sga[b])
                pltpu.async_copy(b_hbm.at[idxs.at[c + NB]], bufb.at[b], sgb[b])

        def body(i, carry):
            for b in range(NB):
                c = i * NB + b
                step(c, b, w * EW + c * K)
            return carry

        lax.fori_loop(0, FULL, body, 0)
        for c in TAIL:
            step(c, c % NB, w * EW + c * K)
        base = w * EW
        for b in range(NB):
            pltpu.make_async_copy(bufa.at[b], ad_hbm.at[pl.ds(base, K)],
                                  sta[b]).wait()
            pltpu.make_async_copy(bufb.at[b], bs_hbm.at[pl.ds(base, K)],
                                  stb[b]).wait()

    return gk(A, B, dstr, srcr)


def _scatter_add(p, dstr, zeros):
    E, H = p.shape
    _, CH, _ = dstr.shape
    EW = CH * K
    NZ = NP // NS

    NB = 2
    FULL = CH // NB
    TAIL = range(FULL * NB, CH)

    @functools.partial(
        pl.kernel,
        out_type=jax.ShapeDtypeStruct((NC, NP, H), jnp.float32),
        mesh=plsc.VectorSubcoreMesh(core_axis_name="c", subcore_axis_name="s"),
        scratch_types=[pltpu.VMEM((CH, K), jnp.int32),
                       pltpu.VMEM((NB, K, H), jnp.float32),
                       pltpu.VMEM_SHARED((NP, H), jnp.float32)]
                      + [pltpu.SemaphoreType.DMA] * NB,
    )
    def sk(p_hbm, d_hbm, z_hbm, s_out, idx, rows, acc, *sl):
        cid = lax.axis_index("c")
        sid = lax.axis_index("s")
        w = sid * NC + cid
        pltpu.sync_copy(d_hbm.at[w], idx)
        pltpu.sync_copy(z_hbm.at[pl.ds(sid * NZ, NZ)],
                        acc.at[pl.ds(sid * NZ, NZ)])
        for b in range(NB):
            pltpu.async_copy(p_hbm.at[pl.ds(w * EW + b * K, K)], rows.at[b],
                             sl[b])
        plsc.subcore_barrier()

        def step(c, b):
            pltpu.make_async_copy(p_hbm.at[pl.ds(w * EW, K)], rows.at[b],
                                  sl[b]).wait()
            pltpu.sync_copy(rows.at[b], acc.at[idx.at[c]], add=True)

            @pl.when(c + NB < CH)
            def _():
                pltpu.async_copy(p_hbm.at[pl.ds(w * EW + (c + NB) * K, K)],
                                 rows.at[b], sl[b])

        def body(i, carry):
            for b in range(NB):
                step(i * NB + b, b)
            return carry

        lax.fori_loop(0, FULL, body, 0)
        for c in TAIL:
            step(c, c % NB)
        plsc.subcore_barrier()
        pltpu.sync_copy(acc.at[pl.ds(sid * NZ, NZ)],
                        s_out.at[cid, pl.ds(sid * NZ, NZ)])

    return sk(p, dstr, zeros)


def _count(dstr, ones, zeros):
    _, CH, _ = dstr.shape
    H = ones.shape[1]
    NZ = NP // NS

    @functools.partial(
        pl.kernel,
        out_type=jax.ShapeDtypeStruct((NC, NP, H), jnp.float32),
        mesh=plsc.VectorSubcoreMesh(core_axis_name="c", subcore_axis_name="s"),
        scratch_types=[pltpu.VMEM((CH, K), jnp.int32),
                       pltpu.VMEM((K, H), jnp.float32),
                       pltpu.VMEM_SHARED((NP, H), jnp.float32),
                       pltpu.SemaphoreType.DMA],
    )
    def ck(d_hbm, ones_hbm, z_hbm, c_out, idx, ones_v, acc, sem):
        cid = lax.axis_index("c")
        sid = lax.axis_index("s")
        pltpu.sync_copy(d_hbm.at[sid * NC + cid], idx)
        pltpu.sync_copy(ones_hbm, ones_v)
        pltpu.sync_copy(z_hbm.at[pl.ds(sid * NZ, NZ)],
                        acc.at[pl.ds(sid * NZ, NZ)])
        plsc.subcore_barrier()
        W = 8

        def body(c, carry):
            pltpu.async_copy(ones_v, acc.at[idx.at[c]], sem, add=True)

            @pl.when(c >= W)
            def _():
                pltpu.make_async_copy(ones_v, acc.at[idx.at[0]], sem).wait()

            return carry

        lax.fori_loop(0, CH, body, 0)
        for _ in range(min(W, CH)):
            pltpu.make_async_copy(ones_v, acc.at[idx.at[0]], sem).wait()
        plsc.subcore_barrier()
        pltpu.sync_copy(acc.at[pl.ds(sid * NZ, NZ)],
                        c_out.at[cid, pl.ds(sid * NZ, NZ)])

    return ck(dstr, ones, zeros)


def _full(shape):
    return pl.BlockSpec(shape, lambda *i: tuple(0 for _ in shape))


def _block_size(E):
    for g in range(24, 201):
        if E % g == 0 and (E // g) % 8 == 0 and E // g <= 6720:
            return E // g
    return 8


def _edge0(xd, xs, dts2, W1a, W1b, W1c, b1, W2, b2, freq, phase):
    E, D = xd.shape
    h = W1a.shape[1]
    TD = W1c.shape[0]
    H = W2.shape[1]
    BE = _block_size(E)
    bf = jnp.bfloat16

    def body(xd_ref, xs_ref, dt_ref, wa_ref, wb_ref, wc_ref, b1_ref, w2_ref,
             b2_ref, f_ref, ph_ref, o_ref):
        te = jnp.cos(dt_ref[...] * f_ref[...] + ph_ref[...])
        pre = (jnp.dot(xd_ref[...].astype(bf), wa_ref[...].astype(bf),
                       preferred_element_type=jnp.float32)
               + jnp.dot(xs_ref[...].astype(bf), wb_ref[...].astype(bf),
                         preferred_element_type=jnp.float32)
               + jnp.dot(te.astype(bf), wc_ref[...].astype(bf),
                         preferred_element_type=jnp.float32)
               + b1_ref[...])
        z = jnp.dot(jax.nn.relu(pre).astype(bf), w2_ref[...].astype(bf),
                    preferred_element_type=jnp.float32) + b2_ref[...]
        o_ref[...] = jnp.tanh(z)

    return pl.pallas_call(
        body,
        grid=(E // BE,),
        in_specs=[pl.BlockSpec((BE, D), lambda i: (i, 0)),
                  pl.BlockSpec((BE, D), lambda i: (i, 0)),
                  pl.BlockSpec((BE, 1), lambda i: (i, 0)),
                  _full((D, h)), _full((D, h)), _full((TD, h)), _full((1, h)),
                  _full((h, H)), _full((1, H)),
                  _full((1, TD)), _full((1, TD))],
        out_specs=pl.BlockSpec((BE, H), lambda i: (i, 0)),
        out_shape=jax.ShapeDtypeStruct((E, H), jnp.float32),
    )(xd, xs, dts2, W1a, W1b, W1c, b1.reshape(1, h), W2, b2.reshape(1, H),
      freq.reshape(1, TD), phase.reshape(1, TD))


def _edge_smp(Ad, Bs, W2, b2):
    E, h = Ad.shape
    H = W2.shape[1]
    BE = _block_size(E)
    bf = jnp.bfloat16

    def body(a_ref, b_ref, w2_ref, b2_ref, o_ref):
        pre = a_ref[...] + b_ref[...]
        z = jnp.dot(jax.nn.relu(pre).astype(bf), w2_ref[...].astype(bf),
                    preferred_element_type=jnp.float32) + b2_ref[...]
        o_ref[...] = jnp.tanh(z)

    return pl.pallas_call(
        body,
        grid=(E // BE,),
        in_specs=[pl.BlockSpec((BE, h), lambda i: (i, 0)),
                  pl.BlockSpec((BE, h), lambda i: (i, 0)),
                  _full((h, H)), _full((1, H))],
        out_specs=pl.BlockSpec((BE, H), lambda i: (i, 0)),
        out_shape=jax.ShapeDtypeStruct((E, H), jnp.float32),
    )(Ad, Bs, W2, b2.reshape(1, H))


def _factor(s_refs, c_ref):
    S = sum(r[0] + r[1] for r in s_refs)
    cnt = c_ref[0, :, 0:1] + c_ref[1, :, 0:1]
    return jnp.where(cnt > 0, 2.0 * S / jnp.maximum(cnt, 1.0), 1.0)


def _upd_pre(h, Ss, C, postW, postb, Wa, Wb, b1, bn):
    N, D = h.shape
    h2 = Wa.shape[1]
    R = 1000
    NS_ = len(Ss)

    def body(*refs):
        (h_ref, *s_refs), rest = refs[:1 + NS_], refs[1 + NS_:]
        (c_ref, pw_ref, pb_ref, wa_ref, wb_ref, b_ref,
         hn_ref, a_ref, bo_ref) = rest
        g = h_ref[...] * _factor(s_refs, c_ref)
        if bn:
            hn = jax.nn.relu(g * INV_BN)
        else:
            hn = jax.nn.relu(jnp.dot(g, pw_ref[...],
                                     preferred_element_type=jnp.float32)
                             + pb_ref[...])
        hn_ref[...] = hn
        a_ref[...] = jnp.dot(hn, wa_ref[...],
                             preferred_element_type=jnp.float32) + b_ref[...]
        bo_ref[...] = jnp.dot(hn, wb_ref[...],
                              preferred_element_type=jnp.float32)

    H = postW.shape[1]
    return pl.pallas_call(
        body,
        grid=(N // R,),
        in_specs=[pl.BlockSpec((R, D), lambda i: (i, 0))]
                 + [pl.BlockSpec((NC, R, 128), lambda i: (0, i, 0))
                    for _ in range(NS_ + 1)]
                 + [_full((D, H)), _full((1, H)),
                    _full((H, h2)), _full((H, h2)), _full((1, h2))],
        out_specs=[pl.BlockSpec((R, H), lambda i: (i, 0)),
                   pl.BlockSpec((R, h2), lambda i: (i, 0)),
                   pl.BlockSpec((R, h2), lambda i: (i, 0))],
        out_shape=[jax.ShapeDtypeStruct((N, H), jnp.float32),
                   jax.ShapeDtypeStruct((N, h2), jnp.float32),
                   jax.ShapeDtypeStruct((N, h2), jnp.float32)],
    )(h, *Ss, C, postW, postb.reshape(1, H), Wa, Wb, b1.reshape(1, h2))


def _clf(hs, Ss, Cs, W1, b1, W2, b2, W3, b3):
    Bsz, D = hs.shape
    H1 = W1.shape[1]
    H2 = W2.shape[1]
    NS_ = len(Ss)

    def body(*refs):
        (h_ref, *s_refs), rest = refs[:1 + NS_], refs[1 + NS_:]
        (c_ref, w1_ref, b1_ref, w2_ref, b2_ref, w3_ref, b3_ref, o_ref) = rest
        g = h_ref[...] * _factor(s_refs, c_ref)
        h3 = jax.nn.relu(g * INV_BN)
        z = jnp.dot(h3, w1_ref[...], preferred_element_type=jnp.float32)
        z = jax.nn.relu((z + b1_ref[...]) * INV_BN)
        z = jnp.dot(z, w2_ref[...], preferred_element_type=jnp.float32)
        z = jax.nn.relu((z + b2_ref[...]) * INV_BN)
        o_ref[...] = jnp.dot(z, w3_ref[...],
                             preferred_element_type=jnp.float32) + b3_ref[...]

    return pl.pallas_call(
        body,
        in_specs=[_full((Bsz, D))]
                 + [_full((NC, Bsz, 128)) for _ in range(NS_ + 1)]
                 + [_full((D, H1)), _full((1, H1)), _full((H1, H2)),
                    _full((1, H2)), _full((H2, 1)), _full((1, 1))],
        out_specs=_full((Bsz, 1)),
        out_shape=jax.ShapeDtypeStruct((Bsz, 1), jnp.float32),
    )(hs, *Ss, Cs, W1, b1.reshape(1, H1), W2, b2.reshape(1, H2), W3,
      b3.reshape(1, 1))


def kernel(x, edge_dts, basis_freq, phase, tmp_W1, tmp_b1, tmp_W2, tmp_b2,
           proj_W, proj_b, smp0_W1, smp0_b1, smp0_W2, smp0_b2, smp1_W1,
           smp1_b1, smp1_W2, smp1_b2, clf_W1, clf_b1, clf_W2, clf_b2, clf_W3,
           clf_b3, edge_index, batch_size):
    N, D = x.shape
    E = edge_index.shape[1]
    H = proj_W.shape[1]
    Bsz = 1024
    EW = E // NW
    CH = EW // K
    NSH = 3
    splits = []
    lo = 0
    for i in range(NSH):
        n = CH // NSH + (1 if i < CH % NSH else 0)
        splits.append((lo, n))
        lo += n

    dstr = edge_index[1].reshape(NW, CH, K)
    srcr = edge_index[0].reshape(NW, CH, K)
    dtsr = edge_dts.reshape(NW, CH, K)
    shards = []
    for lo, n in splits:
        dsh = dstr[:, lo:lo + n]
        ssh = srcr[:, lo:lo + n]
        dts = dtsr[:, lo:lo + n].reshape(NW * n * K, 1)
        shards.append((dsh, ssh, dts))
    z128 = jnp.zeros((NP, 128), jnp.float32)
    ones128 = jnp.ones((K, 128), jnp.float32)

    W1a, W1b, W1c = tmp_W1[:D], tmp_W1[D:2 * D], tmp_W1[2 * D:]
    Ss = []
    for dsh, ssh, dts in shards:
        xd, xs = _gather_pair(x, x, dsh, ssh)
        p = _edge0(xd, xs, dts, W1a, W1b, W1c, tmp_b1, tmp_W2, tmp_b2,
                   basis_freq, phase)
        Ss.append(_scatter_add(p, dsh, z128))
    C = _count(dstr, ones128, z128)
    h1, A1, B1 = _upd_pre(x, Ss, C, proj_W, proj_b, smp0_W1[:H],
                          smp0_W1[H:], smp0_b1, bn=False)

    Ss = []
    for dsh, ssh, _ in shards:
        Ad, Bs = _gather_pair(A1, B1, dsh, ssh)
        p = _edge_smp(Ad, Bs, smp0_W2, smp0_b2)
        Ss.append(_scatter_add(p, dsh, z128))
    h2, A2, B2 = _upd_pre(h1, Ss, C, proj_W, proj_b, smp1_W1[:H],
                          smp1_W1[H:], smp1_b1, bn=True)

    Ss = []
    for dsh, ssh, _ in shards:
        Ad, Bs = _gather_pair(A2, B2, dsh, ssh)
        p = _edge_smp(Ad, Bs, smp1_W2, smp1_b2)
        Ss.append(_scatter_add(p, dsh, z128))
    return _clf(h2[:Bsz], [S[:, :Bsz] for S in Ss], C[:, :Bsz], clf_W1,
                clf_b1, clf_W2, clf_b2, clf_W3, clf_b3)


# speedup vs baseline: 1.0108x; 1.0108x over previous
import functools

import jax
import jax.numpy as jnp
from jax import lax
from jax.experimental import pallas as pl
from jax.experimental.pallas import tpu as pltpu
from jax.experimental.pallas import tpu_sc as plsc

NC, NS = 2, 16
NW = NC * NS
K = 80
NP = 10240
INV_BN = 1.0 / (1.0 + 1e-5) ** 0.5


def _gather_pair(A, B, dstr, srcr):
    N, h = A.shape
    dt = A.dtype
    _, CH, _ = dstr.shape
    E = NW * CH * K
    EW = CH * K

    NB = 5
    FULL = CH // NB
    TAIL = range(FULL * NB, CH)

    @functools.partial(
        pl.kernel,
        out_type=[jax.ShapeDtypeStruct((E, h), dt),
                  jax.ShapeDtypeStruct((E, h), dt)],
        mesh=plsc.VectorSubcoreMesh(core_axis_name="c", subcore_axis_name="s"),
        scratch_types=[pltpu.VMEM((CH, K), jnp.int32),
                       pltpu.VMEM((CH, K), jnp.int32),
                       pltpu.VMEM((NB, K, h), dt),
                       pltpu.VMEM((NB, K, h), dt)]
                      + [pltpu.SemaphoreType.DMA] * (4 * NB),
    )
    def gk(a_hbm, b_hbm, d_hbm, s_hbm, ad_hbm, bs_hbm, idxd, idxs, bufa, bufb,
           *sems):
        sga, sgb, sta, stb = (sems[:NB], sems[NB:2 * NB], sems[2 * NB:3 * NB],
                              sems[3 * NB:])
        w = lax.axis_index("s") * NC + lax.axis_index("c")
        pltpu.sync_copy(d_hbm.at[w], idxd)
        pltpu.sync_copy(s_hbm.at[w], idxs)
        for b in range(NB):
            pltpu.async_copy(a_hbm.at[idxd.at[b]], bufa.at[b], sga[b])
            pltpu.async_copy(b_hbm.at[idxs.at[b]], bufb.at[b], sgb[b])

        def step(c, b, base):
            pltpu.make_async_copy(a_hbm.at[idxd.at[c]], bufa.at[b],
                                  sga[b]).wait()
            pltpu.make_async_copy(b_hbm.at[idxs.at[c]], bufb.at[b],
                                  sgb[b]).wait()
            pltpu.async_copy(bufa.at[b], ad_hbm.at[pl.ds(base, K)], sta[b])
            pltpu.async_copy(bufb.at[b], bs_hbm.at[pl.ds(base, K)], stb[b])

            @pl.when(c + NB < CH)
            def _():
                pltpu.make_async_copy(bufa.at[b], ad_hbm.at[pl.ds(base, K)],
                                      sta[b]).wait()
                pltpu.make_async_copy(bufb.at[b], bs_hbm.at[pl.ds(base, K)],
                                      stb[b]).wait()
                pltpu.async_copy(a_hbm.at[idxd.at[c + NB]], bufa.at[b], sga[b])
                pltpu.async_copy(b_hbm.at[idxs.at[c + NB]], bufb.at[b], sgb[b])

        def body(i, carry):
            for b in range(NB):
                c = i * NB + b
                step(c, b, w * EW + c * K)
            return carry

        lax.fori_loop(0, FULL, body, 0)
        for c in TAIL:
            step(c, c % NB, w * EW + c * K)
        base = w * EW
        for b in range(NB):
            pltpu.make_async_copy(bufa.at[b], ad_hbm.at[pl.ds(base, K)],
                                  sta[b]).wait()
            pltpu.make_async_copy(bufb.at[b], bs_hbm.at[pl.ds(base, K)],
                                  stb[b]).wait()

    return gk(A, B, dstr, srcr)


def _scatter_add(p, dstr, zeros):
    E, H = p.shape
    _, CH, _ = dstr.shape
    EW = CH * K
    NZ = NP // NS

    NB = 2
    FULL = CH // NB
    TAIL = range(FULL * NB, CH)

    @functools.partial(
        pl.kernel,
        out_type=jax.ShapeDtypeStruct((NC, NP, H), jnp.float32),
        mesh=plsc.VectorSubcoreMesh(core_axis_name="c", subcore_axis_name="s"),
        scratch_types=[pltpu.VMEM((CH, K), jnp.int32),
                       pltpu.VMEM((NB, K, H), jnp.float32),
                       pltpu.VMEM_SHARED((NP, H), jnp.float32)]
                      + [pltpu.SemaphoreType.DMA] * NB,
    )
    def sk(p_hbm, d_hbm, z_hbm, s_out, idx, rows, acc, *sl):
        cid = lax.axis_index("c")
        sid = lax.axis_index("s")
        w = sid * NC + cid
        pltpu.sync_copy(d_hbm.at[w], idx)
        pltpu.sync_copy(z_hbm.at[pl.ds(sid * NZ, NZ)],
                        acc.at[pl.ds(sid * NZ, NZ)])
        for b in range(NB):
            pltpu.async_copy(p_hbm.at[pl.ds(w * EW + b * K, K)], rows.at[b],
                             sl[b])
        plsc.subcore_barrier()

        def step(c, b):
            pltpu.make_async_copy(p_hbm.at[pl.ds(w * EW, K)], rows.at[b],
                                  sl[b]).wait()
            pltpu.sync_copy(rows.at[b], acc.at[idx.at[c]], add=True)

            @pl.when(c + NB < CH)
            def _():
                pltpu.async_copy(p_hbm.at[pl.ds(w * EW + (c + NB) * K, K)],
                                 rows.at[b], sl[b])

        def body(i, carry):
            for b in range(NB):
                step(i * NB + b, b)
            return carry

        lax.fori_loop(0, FULL, body, 0)
        for c in TAIL:
            step(c, c % NB)
        plsc.subcore_barrier()
        pltpu.sync_copy(acc.at[pl.ds(sid * NZ, NZ)],
                        s_out.at[cid, pl.ds(sid * NZ, NZ)])

    return sk(p, dstr, zeros)


def _count(dstr, ones, zeros):
    _, CH, _ = dstr.shape
    H = ones.shape[1]
    NZ = NP // NS

    @functools.partial(
        pl.kernel,
        out_type=jax.ShapeDtypeStruct((NC, NP, H), jnp.float32),
        mesh=plsc.VectorSubcoreMesh(core_axis_name="c", subcore_axis_name="s"),
        scratch_types=[pltpu.VMEM((CH, K), jnp.int32),
                       pltpu.VMEM((K, H), jnp.float32),
                       pltpu.VMEM_SHARED((NP, H), jnp.float32),
                       pltpu.SemaphoreType.DMA],
    )
    def ck(d_hbm, ones_hbm, z_hbm, c_out, idx, ones_v, acc, sem):
        cid = lax.axis_index("c")
        sid = lax.axis_index("s")
        pltpu.sync_copy(d_hbm.at[sid * NC + cid], idx)
        pltpu.sync_copy(ones_hbm, ones_v)
        pltpu.sync_copy(z_hbm.at[pl.ds(sid * NZ, NZ)],
                        acc.at[pl.ds(sid * NZ, NZ)])
        plsc.subcore_barrier()
        W = 8

        def body(c, carry):
            pltpu.async_copy(ones_v, acc.at[idx.at[c]], sem, add=True)

            @pl.when(c >= W)
            def _():
                pltpu.make_async_copy(ones_v, acc.at[idx.at[0]], sem).wait()

            return carry

        lax.fori_loop(0, CH, body, 0)
        for _ in range(min(W, CH)):
            pltpu.make_async_copy(ones_v, acc.at[idx.at[0]], sem).wait()
        plsc.subcore_barrier()
        pltpu.sync_copy(acc.at[pl.ds(sid * NZ, NZ)],
                        c_out.at[cid, pl.ds(sid * NZ, NZ)])

    return ck(dstr, ones, zeros)


def _full(shape):
    return pl.BlockSpec(shape, lambda *i: tuple(0 for _ in shape))


def _block_size(E):
    for g in range(24, 201):
        if E % g == 0 and (E // g) % 8 == 0 and E // g <= 6720:
            return E // g
    return 8


def _edge0(xd, xs, dts2, W1a, W1b, W1c, b1, W2, b2, freq, phase):
    E, D = xd.shape
    h = W1a.shape[1]
    TD = W1c.shape[0]
    H = W2.shape[1]
    BE = _block_size(E)
    bf = jnp.bfloat16

    def body(xd_ref, xs_ref, dt_ref, wa_ref, wb_ref, wc_ref, b1_ref, w2_ref,
             b2_ref, f_ref, ph_ref, o_ref):
        te = jnp.cos(dt_ref[...] * f_ref[...] + ph_ref[...])
        pre = (jnp.dot(xd_ref[...].astype(bf), wa_ref[...].astype(bf),
                       preferred_element_type=jnp.float32)
               + jnp.dot(xs_ref[...].astype(bf), wb_ref[...].astype(bf),
                         preferred_element_type=jnp.float32)
               + jnp.dot(te.astype(bf), wc_ref[...].astype(bf),
                         preferred_element_type=jnp.float32)
               + b1_ref[...])
        z = jnp.dot(jax.nn.relu(pre).astype(bf), w2_ref[...].astype(bf),
                    preferred_element_type=jnp.float32) + b2_ref[...]
        o_ref[...] = jnp.tanh(z)

    return pl.pallas_call(
        body,
        grid=(E // BE,),
        in_specs=[pl.BlockSpec((BE, D), lambda i: (i, 0)),
                  pl.BlockSpec((BE, D), lambda i: (i, 0)),
                  pl.BlockSpec((BE, 1), lambda i: (i, 0)),
                  _full((D, h)), _full((D, h)), _full((TD, h)), _full((1, h)),
                  _full((h, H)), _full((1, H)),
                  _full((1, TD)), _full((1, TD))],
        out_specs=pl.BlockSpec((BE, H), lambda i: (i, 0)),
        out_shape=jax.ShapeDtypeStruct((E, H), jnp.float32),
    )(xd, xs, dts2, W1a, W1b, W1c, b1.reshape(1, h), W2, b2.reshape(1, H),
      freq.reshape(1, TD), phase.reshape(1, TD))


def _edge_smp(Ad, Bs, W2, b2):
    E, h = Ad.shape
    H = W2.shape[1]
    BE = _block_size(E)
    bf = jnp.bfloat16

    def body(a_ref, b_ref, w2_ref, b2_ref, o_ref):
        pre = a_ref[...] + b_ref[...]
        z = jnp.dot(jax.nn.relu(pre).astype(bf), w2_ref[...].astype(bf),
                    preferred_element_type=jnp.float32) + b2_ref[...]
        o_ref[...] = jnp.tanh(z)

    return pl.pallas_call(
        body,
        grid=(E // BE,),
        in_specs=[pl.BlockSpec((BE, h), lambda i: (i, 0)),
                  pl.BlockSpec((BE, h), lambda i: (i, 0)),
                  _full((h, H)), _full((1, H))],
        out_specs=pl.BlockSpec((BE, H), lambda i: (i, 0)),
        out_shape=jax.ShapeDtypeStruct((E, H), jnp.float32),
    )(Ad, Bs, W2, b2.reshape(1, H))


def _factor(s_refs, c_ref):
    S = sum(r[0] + r[1] for r in s_refs)
    cnt = c_ref[0, :, 0:1] + c_ref[1, :, 0:1]
    return jnp.where(cnt > 0, 2.0 * S / jnp.maximum(cnt, 1.0), 1.0)


def _upd_pre(h, Ss, C, postW, postb, Wa, Wb, b1, bn):
    N, D = h.shape
    h2 = Wa.shape[1]
    R = 1000
    NS_ = len(Ss)

    def body(*refs):
        (h_ref, *s_refs), rest = refs[:1 + NS_], refs[1 + NS_:]
        (c_ref, pw_ref, pb_ref, wa_ref, wb_ref, b_ref,
         hn_ref, a_ref, bo_ref) = rest
        g = h_ref[...] * _factor(s_refs, c_ref)
        if bn:
            hn = jax.nn.relu(g * INV_BN)
        else:
            hn = jax.nn.relu(jnp.dot(g, pw_ref[...],
                                     preferred_element_type=jnp.float32)
                             + pb_ref[...])
        hn_ref[...] = hn
        a_ref[...] = jnp.dot(hn, wa_ref[...],
                             preferred_element_type=jnp.float32) + b_ref[...]
        bo_ref[...] = jnp.dot(hn, wb_ref[...],
                              preferred_element_type=jnp.float32)

    H = postW.shape[1]
    return pl.pallas_call(
        body,
        grid=(N // R,),
        in_specs=[pl.BlockSpec((R, D), lambda i: (i, 0))]
                 + [pl.BlockSpec((NC, R, 128), lambda i: (0, i, 0))
                    for _ in range(NS_ + 1)]
                 + [_full((D, H)), _full((1, H)),
                    _full((H, h2)), _full((H, h2)), _full((1, h2))],
        out_specs=[pl.BlockSpec((R, H), lambda i: (i, 0)),
                   pl.BlockSpec((R, h2), lambda i: (i, 0)),
                   pl.BlockSpec((R, h2), lambda i: (i, 0))],
        out_shape=[jax.ShapeDtypeStruct((N, H), jnp.float32),
                   jax.ShapeDtypeStruct((N, h2), jnp.float32),
                   jax.ShapeDtypeStruct((N, h2), jnp.float32)],
    )(h, *Ss, C, postW, postb.reshape(1, H), Wa, Wb, b1.reshape(1, h2))


def _clf(hs, Ss, Cs, W1, b1, W2, b2, W3, b3):
    Bsz, D = hs.shape
    H1 = W1.shape[1]
    H2 = W2.shape[1]
    NS_ = len(Ss)

    def body(*refs):
        (h_ref, *s_refs), rest = refs[:1 + NS_], refs[1 + NS_:]
        (c_ref, w1_ref, b1_ref, w2_ref, b2_ref, w3_ref, b3_ref, o_ref) = rest
        g = h_ref[...] * _factor(s_refs, c_ref)
        h3 = jax.nn.relu(g * INV_BN)
        z = jnp.dot(h3, w1_ref[...], preferred_element_type=jnp.float32)
        z = jax.nn.relu((z + b1_ref[...]) * INV_BN)
        z = jnp.dot(z, w2_ref[...], preferred_element_type=jnp.float32)
        z = jax.nn.relu((z + b2_ref[...]) * INV_BN)
        o_ref[...] = jnp.dot(z, w3_ref[...],
                             preferred_element_type=jnp.float32) + b3_ref[...]

    return pl.pallas_call(
        body,
        in_specs=[_full((Bsz, D))]
                 + [_full((NC, Bsz, 128)) for _ in range(NS_ + 1)]
                 + [_full((D, H1)), _full((1, H1)), _full((H1, H2)),
                    _full((1, H2)), _full((H2, 1)), _full((1, 1))],
        out_specs=_full((Bsz, 1)),
        out_shape=jax.ShapeDtypeStruct((Bsz, 1), jnp.float32),
    )(hs, *Ss, Cs, W1, b1.reshape(1, H1), W2, b2.reshape(1, H2), W3,
      b3.reshape(1, 1))


def kernel(x, edge_dts, basis_freq, phase, tmp_W1, tmp_b1, tmp_W2, tmp_b2,
           proj_W, proj_b, smp0_W1, smp0_b1, smp0_W2, smp0_b2, smp1_W1,
           smp1_b1, smp1_W2, smp1_b2, clf_W1, clf_b1, clf_W2, clf_b2, clf_W3,
           clf_b3, edge_index, batch_size):
    N, D = x.shape
    E = edge_index.shape[1]
    H = proj_W.shape[1]
    Bsz = 1024
    EW = E // NW
    CH = EW // K
    NSH = 2
    splits = []
    lo = 0
    for i in range(NSH):
        n = CH // NSH + (1 if i < CH % NSH else 0)
        splits.append((lo, n))
        lo += n

    dstr = edge_index[1].reshape(NW, CH, K)
    srcr = edge_index[0].reshape(NW, CH, K)
    dtsr = edge_dts.reshape(NW, CH, K)
    shards = []
    for lo, n in splits:
        dsh = dstr[:, lo:lo + n]
        ssh = srcr[:, lo:lo + n]
        dts = dtsr[:, lo:lo + n].reshape(NW * n * K, 1)
        shards.append((dsh, ssh, dts))
    z128 = jnp.zeros((NP, 128), jnp.float32)
    ones128 = jnp.ones((K, 128), jnp.float32)

    W1a, W1b, W1c = tmp_W1[:D], tmp_W1[D:2 * D], tmp_W1[2 * D:]
    Ss = []
    for dsh, ssh, dts in shards:
        xd, xs = _gather_pair(x, x, dsh, ssh)
        p = _edge0(xd, xs, dts, W1a, W1b, W1c, tmp_b1, tmp_W2, tmp_b2,
                   basis_freq, phase)
        Ss.append(_scatter_add(p, dsh, z128))
    C = _count(dstr, ones128, z128)
    h1, A1, B1 = _upd_pre(x, Ss, C, proj_W, proj_b, smp0_W1[:H],
                          smp0_W1[H:], smp0_b1, bn=False)

    Ss = []
    for dsh, ssh, _ in shards:
        Ad, Bs = _gather_pair(A1, B1, dsh, ssh)
        p = _edge_smp(Ad, Bs, smp0_W2, smp0_b2)
        Ss.append(_scatter_add(p, dsh, z128))
    h2, A2, B2 = _upd_pre(h1, Ss, C, proj_W, proj_b, smp1_W1[:H],
                          smp1_W1[H:], smp1_b1, bn=True)

    Ss = []
    for dsh, ssh, _ in shards:
        Ad, Bs = _gather_pair(A2, B2, dsh, ssh)
        p = _edge_smp(Ad, Bs, smp1_W2, smp1_b2)
        Ss.append(_scatter_add(p, dsh, z128))
    return _clf(h2[:Bsz], [S[:, :Bsz] for S in Ss], C[:, :Bsz], clf_W1,
                clf_b1, clf_W2, clf_b2, clf_W3, clf_b3)
